# baseline (speedup 1.0000x reference)
import jax
import jax.numpy as jnp
from jax import lax
from jax.experimental import pallas as pl
from jax.experimental.pallas import tpu as pltpu
from jax.experimental.pallas import tpu_sc as plsc

MAXLEN = 8192
D = 1024
BLANK0, BLANK1, BLANK2 = 50257, 50258, 50259

NC, NS = 2, 16
NW = NC * NS
TPW = 1024
WPR = MAXLEN // TPW
NCH = TPW // 16
G = 16
NBUF = 7


def _is_blank_i32(v):
    d0 = jnp.abs(v - BLANK0)
    d1 = jnp.abs(v - BLANK1)
    d2 = jnp.abs(v - BLANK2)
    return 1 - jnp.minimum(jnp.minimum(jnp.minimum(d0, d1), d2), 1)


def _vgather(s, idx):
    return lax.gather(
        s, idx[:, None],
        lax.GatherDimensionNumbers(
            offset_dims=(), collapsed_slice_dims=(0,), start_index_map=(0,)),
        (1,), mode=lax.GatherScatterMode.PROMISE_IN_BOUNDS)


def _body(x_hbm, w_hbm, out_hbm, x_v, idx_v, *rest):
    bufs = rest[:NBUF]
    gsems = rest[NBUF:2 * NBUF]
    osems = rest[2 * NBUF:]
    cid = lax.axis_index("c")
    sid = lax.axis_index("s")
    wid = sid * NC + cid
    b = wid // WPR
    base_t = (wid % WPR) * TPW

    iot = lax.iota(jnp.int32, 16)

    def _cumsum16(s):
        for k in (1, 2, 4, 8):
            sh = _vgather(s, jnp.maximum(iot - k, 0))
            msk = jnp.minimum(jnp.maximum(iot - (k - 1), 0), 1)
            s = s + sh * msk
        return s

    def _bcast_last(s):
        return _vgather(s, jnp.full((16,), 15, jnp.int32))

    pltpu.sync_copy(x_hbm.at[b], x_v)

    def pc_body(c, a):
        for u in range(4):
            a = a + _is_blank_i32(x_v[pl.ds((c * 4 + u) * 16, 16)])
        return a

    acc = lax.fori_loop(0, base_t // 64, pc_body,
                        jnp.zeros((16,), jnp.int32))
    prefix_vec = _bcast_last(_cumsum16(acc))

    def ix_body(c, carry):
        blk = _is_blank_i32(x_v[pl.ds(base_t + c * 16, 16)])
        cs = _cumsum16(blk)
        pos = base_t + c * 16 + iot - carry - cs
        idx_v[c, :] = jnp.clip(pos, 0, MAXLEN - 1)
        return carry + _bcast_last(cs)

    out_e0 = wid * TPW * D

    def _start_gather(c, s):
        b0 = idx_v[c, :][0]
        pltpu.async_copy(w_hbm.at[pl.ds(b0 * D, G * D)], bufs[s],
                         gsems[s])

    def _wait_gather(s):
        pltpu.make_async_copy(w_hbm.at[pl.ds(0, G * D)], bufs[s],
                              gsems[s]).wait()

    def _fix_chunk(c, s):
        vv = idx_v[c, :]
        b0 = vv[0]

        @pl.when(vv[15] - b0 != 15)
        def _():
            for j in range(G - 1, 0, -1):
                src = vv[j] - b0

                @pl.when(src != j)
                def _():
                    def cp(q, _q):
                        bufs[s][pl.ds(j * D + q * 16, 16)] = (
                            bufs[s][pl.ds(src * D + q * 16, 16)])
                        return 0

                    lax.fori_loop(0, D // 16, cp, 0)

    def _start_out(c, s):
        pltpu.async_copy(bufs[s],
                         out_hbm.at[pl.ds(out_e0 + c * G * D, G * D)],
                         osems[s])

    def _wait_out(s):
        pltpu.make_async_copy(bufs[s], out_hbm.at[pl.ds(out_e0, G * D)],
                              osems[s]).wait()

    carry = prefix_vec
    for s in range(NBUF):
        carry = ix_body(s, carry)
        _start_gather(s, s)

    lax.fori_loop(NBUF, NCH, ix_body, carry)

    def ring_round(r, _):
        c0 = r * NBUF
        for s in range(NBUF):
            @pl.when(c0 + s < NCH)
            def _():
                _wait_gather(s)
                _fix_chunk(c0 + s, s)
                _start_out(c0 + s, s)
        for s in range(NBUF):
            @pl.when(c0 + s + NBUF < NCH)
            def _():
                _wait_out(s)
                _start_gather(c0 + s + NBUF, s)
        return 0

    lax.fori_loop(0, (NCH + NBUF - 1) // NBUF, ring_round, 0)

    for s in range(min(NBUF, NCH)):
        _wait_out(s)


@jax.jit
def kernel(x, weight):
    bsz, seqlen = x.shape
    mesh = plsc.VectorSubcoreMesh(core_axis_name="c", subcore_axis_name="s")
    f = pl.kernel(
        _body,
        out_type=jax.ShapeDtypeStruct((bsz * seqlen * D,), jnp.float32),
        mesh=mesh,
        scratch_types=[
            pltpu.VMEM((MAXLEN,), jnp.int32),
            pltpu.VMEM((NCH, 16), jnp.int32),
        ] + [pltpu.VMEM((G * D,), jnp.float32)] * NBUF
          + [pltpu.SemaphoreType.DMA] * (2 * NBUF),
    )
    out = f(x, weight.reshape(-1))
    return out.reshape(bsz, seqlen, D)


# speedup vs baseline: 2.4428x; 2.4428x over previous
import jax
import jax.numpy as jnp
from jax import lax
from jax.experimental import pallas as pl
from jax.experimental.pallas import tpu as pltpu
from jax.experimental.pallas import tpu_sc as plsc

MAXLEN = 8192
D = 1024
BLANK0, BLANK1, BLANK2 = 50257, 50258, 50259

NC, NS = 2, 16
NW = NC * NS
TPW = 1024
WPR = MAXLEN // TPW
NCH = TPW // 16
G = 16
NBUF = 7


def _is_blank_i32(v):
    d0 = jnp.abs(v - BLANK0)
    d1 = jnp.abs(v - BLANK1)
    d2 = jnp.abs(v - BLANK2)
    return 1 - jnp.minimum(jnp.minimum(jnp.minimum(d0, d1), d2), 1)


def _vgather(s, idx):
    return lax.gather(
        s, idx[:, None],
        lax.GatherDimensionNumbers(
            offset_dims=(), collapsed_slice_dims=(0,), start_index_map=(0,)),
        (1,), mode=lax.GatherScatterMode.PROMISE_IN_BOUNDS)


def _body(x_hbm, w_hbm, out_hbm, x_v, idx_v, bufs, *sems):
    gsems = sems[:NBUF]
    osems = sems[NBUF:]
    cid = lax.axis_index("c")
    sid = lax.axis_index("s")
    wid = sid * NC + cid
    b = wid // WPR
    base_t = (wid % WPR) * TPW

    iot = lax.iota(jnp.int32, 16)

    def _cumsum16(s):
        for k in (1, 2, 4, 8):
            sh = _vgather(s, jnp.maximum(iot - k, 0))
            msk = jnp.minimum(jnp.maximum(iot - (k - 1), 0), 1)
            s = s + sh * msk
        return s

    def _bcast_last(s):
        return _vgather(s, jnp.full((16,), 15, jnp.int32))

    pltpu.sync_copy(x_hbm.at[b], x_v)

    def pc_body(c, a):
        for u in range(4):
            a = a + _is_blank_i32(x_v[pl.ds((c * 4 + u) * 16, 16)])
        return a

    acc = lax.fori_loop(0, base_t // 64, pc_body,
                        jnp.zeros((16,), jnp.int32))
    prefix_vec = _bcast_last(_cumsum16(acc))

    def ix_body(c, carry):
        blk = _is_blank_i32(x_v[pl.ds(base_t + c * 16, 16)])
        cs = _cumsum16(blk)
        pos = base_t + c * 16 + iot - carry - cs
        idx_v[c, :] = jnp.clip(pos, 0, MAXLEN - 1)
        return carry + _bcast_last(cs)

    out_row0 = wid * TPW

    def _start_gather(c, s):
        pltpu.async_copy(w_hbm.at[idx_v.at[c]], bufs.at[s], gsems[s])

    def _wait_gather(s):
        pltpu.make_async_copy(w_hbm.at[pl.ds(0, G)], bufs.at[s],
                              gsems[s]).wait()

    def _start_out(c, s):
        pltpu.async_copy(bufs.at[s],
                         out_hbm.at[pl.ds(out_row0 + c * G, G)], osems[s])

    def _wait_out(s):
        pltpu.make_async_copy(bufs.at[s], out_hbm.at[pl.ds(out_row0, G)],
                              osems[s]).wait()

    carry = prefix_vec
    for s in range(NBUF):
        carry = ix_body(s, carry)
        _start_gather(s, s)

    lax.fori_loop(NBUF, NCH, ix_body, carry)

    def ring_round(r, _):
        c0 = r * NBUF
        for s in range(NBUF):
            @pl.when(c0 + s < NCH)
            def _():
                _wait_gather(s)
                _start_out(c0 + s, s)
        for s in range(NBUF):
            @pl.when(c0 + s + NBUF < NCH)
            def _():
                _wait_out(s)
                _start_gather(c0 + s + NBUF, s)
        return 0

    lax.fori_loop(0, (NCH + NBUF - 1) // NBUF, ring_round, 0)

    for s in range(min(NBUF, NCH)):
        _wait_out(s)


@jax.jit
def kernel(x, weight):
    bsz, seqlen = x.shape
    mesh = plsc.VectorSubcoreMesh(core_axis_name="c", subcore_axis_name="s")
    f = pl.kernel(
        _body,
        out_type=jax.ShapeDtypeStruct((bsz * seqlen, D), jnp.float32),
        mesh=mesh,
        scratch_types=[
            pltpu.VMEM((MAXLEN,), jnp.int32),
            pltpu.VMEM((NCH, 16), jnp.int32),
            pltpu.VMEM((NBUF, G, D), jnp.float32),
        ] + [pltpu.SemaphoreType.DMA] * (2 * NBUF),
    )
    out = f(x, weight)
    return out.reshape(bsz, seqlen, D)
